# Initial kernel scaffold; baseline (speedup 1.0000x reference)
import jax
import jax.numpy as jnp
from jax.experimental import pallas as pl


def kernel(concatenated_node_features, interaction_feature, num_obj, num_relation, object_pairs, Wphi1, bphi1, Wphi2, bphi2, Wpsi1, bpsi1, Wpsi2, bpsi2, Wobj, bobj, Wrel, brel):
    raise NotImplementedError("write your pallas kernel here")


# speedup vs baseline: 4.7135x; 4.7135x over previous
import functools

import jax
import jax.numpy as jnp
from jax import lax
from jax.experimental import pallas as pl
from jax.experimental.pallas import tpu as pltpu
from jax.experimental.pallas import tpu_sc as plsc

_B, _N, _DN, _DE, _H1, _H2, _DOUT, _MAXP, _MAXE = 8, 64, 2048, 1024, 512, 256, 512, 32, 172
_NN = _N * _N
_SLOTP = 192
_L = 16
_NW = 32
_WPS = _NW // _B
_RPW = _SLOTP // _WPS


def _dense_body(x_ref, wphi1_ref, bphi1_ref, wphi2_ref, bphi2_ref,
                wpsi1_ref, bpsi1_ref, wpsi2_ref, bpsi2_ref,
                wobj_ref, bobj_ref, i0_ref, i1_ref, nobj_ref, nrel_ref,
                score_ref, nemb_ref, slot_ref, vmask_ref):
    f32 = jnp.float32
    x = x_ref[...]
    h = jnp.maximum(jnp.dot(x, wphi1_ref[...]) + bphi1_ref[...], 0.0)
    phi = jnp.dot(h, wphi2_ref[...]) + bphi2_ref[...]
    h = jnp.maximum(jnp.dot(x, wpsi1_ref[...]) + bpsi1_ref[...], 0.0)
    psi = jnp.dot(h, wpsi2_ref[...]) + bpsi2_ref[...]
    nemb_ref[...] = jnp.dot(x, wobj_ref[...]) + bobj_ref[...]

    row_i = lax.broadcasted_iota(jnp.int32, (_N, _N), 0)
    col_j = lax.broadcasted_iota(jnp.int32, (_N, _N), 1)
    upper = (row_i <= col_j).astype(f32)
    lows = (col_j < row_i).astype(f32)
    it_node = lax.broadcasted_iota(jnp.int32, (_N, _MAXP), 0)
    it_pair = lax.broadcasted_iota(jnp.int32, (_N, _MAXP), 1)
    iota_s = lax.broadcasted_iota(jnp.int32, (1, _SLOTP), 1).astype(f32)
    dn = (((1,), (1,)), ((), ()))

    for b in range(_B):
        phib = phi[b * _N:(b + 1) * _N]
        psib = psi[b * _N:(b + 1) * _N]
        score = jax.nn.sigmoid(lax.dot_general(phib, psib, dn))
        score_ref[b] = score
        nobj = nobj_ref[b]
        nrel = nrel_ref[b]
        adj = (score > 0.5) & (row_i < nobj) & (col_j < nobj)
        i0b = jnp.broadcast_to(i0_ref[b:b + 1, :], (_N, _MAXP))
        i1b = jnp.broadcast_to(i1_ref[b:b + 1, :], (_N, _MAXP))
        pvalid = it_pair < nrel
        m0 = (it_node == i0b)
        m1 = (it_node == i1b)
        fa = lax.dot_general((m0 & pvalid).astype(f32), m1.astype(f32), dn)
        fb = lax.dot_general((m1 & pvalid).astype(f32), m0.astype(f32), dn)
        adjf = jnp.where(adj | (fa + fb > 0.5), 1.0, 0.0).astype(f32)
        rowcum = jnp.dot(adjf, upper)
        rowtot = rowcum[:, _N - 1:_N]
        prefix = jnp.dot(lows, rowtot)
        rank = prefix + rowcum - 1.0
        validf = (adjf > 0.5) & (rank < float(_MAXE))
        slot_ref[b] = jnp.where(validf, rank, float(_MAXE)).astype(jnp.int32)
        total = jnp.sum(adjf)
        vmask_ref[b:b + 1, :] = (iota_s < total).astype(f32)


def _rel_body(g_ref, wrel_ref, brel_ref, vmask_ref, out_ref):
    out_ref[...] = (jnp.dot(g_ref[...], wrel_ref[...]) + brel_ref[...]) * vmask_ref[...]


@functools.cache
def _sc_gather_fn():
    mesh = plsc.VectorSubcoreMesh(core_axis_name="c", subcore_axis_name="s",
                                  num_cores=2, num_subcores=16)

    @functools.partial(
        pl.kernel,
        out_type=jax.ShapeDtypeStruct((_B * _SLOTP, _DE), jnp.float32),
        mesh=mesh,
        scratch_types=[
            pltpu.VMEM((_NN,), jnp.int32),
            pltpu.VMEM((_SLOTP,), jnp.int32),
            pltpu.VMEM((_RPW,), jnp.int32),
            pltpu.VMEM((_RPW, _DE), jnp.float32),
            pltpu.SemaphoreType.DMA,
        ],
        compiler_params=pltpu.CompilerParams(needs_layout_passes=False),
    )
    def sc_gather(slot_hbm, table_hbm, out_hbm, slot_v, eidx_v, idx_v, rows_v, sem):
        wid = lax.axis_index("s") * 2 + lax.axis_index("c")
        scene = wid // _WPS
        sub = wid % _WPS
        pltpu.sync_copy(slot_hbm.at[scene], slot_v)
        zero = jnp.zeros((_L,), jnp.int32)
        for k in range(_SLOTP // _L):
            eidx_v[pl.ds(k * _L, _L)] = zero
        lane = lax.iota(jnp.int32, _L)

        def scat(e, carry):
            base = e * _L
            chunk = slot_v[pl.ds(base, _L)]
            plsc.store_scatter(eidx_v, [chunk], lane + base)
            return carry

        lax.fori_loop(0, _NN // _L, scat, 0)
        off = jnp.full((_L,), scene * _NN, jnp.int32)
        base = sub * _RPW
        for k in range(_RPW // _L):
            idx_v[pl.ds(k * _L, _L)] = eidx_v[pl.ds(base + k * _L, _L)] + off
        pltpu.async_copy(table_hbm.at[idx_v], rows_v, sem).wait()
        pltpu.sync_copy(rows_v, out_hbm.at[pl.ds(scene * _SLOTP + base, _RPW)])

    return sc_gather


def kernel(concatenated_node_features, interaction_feature, num_obj, num_relation,
           object_pairs, Wphi1, bphi1, Wphi2, bphi2, Wpsi1, bpsi1, Wpsi2, bpsi2,
           Wobj, bobj, Wrel, brel):
    f32 = jnp.float32
    x = concatenated_node_features.reshape(_B * _N, _DN)
    i0 = object_pairs[..., 0].astype(jnp.int32)
    i1 = object_pairs[..., 1].astype(jnp.int32)
    nobj = num_obj.astype(jnp.int32)
    nrel = num_relation.astype(jnp.int32)

    smem = pl.BlockSpec(memory_space=pltpu.SMEM)
    vmem = pl.BlockSpec(memory_space=pltpu.VMEM)
    score, nemb, slot, vmask = pl.pallas_call(
        _dense_body,
        out_shape=[
            jax.ShapeDtypeStruct((_B, _N, _N), f32),
            jax.ShapeDtypeStruct((_B * _N, _DOUT), f32),
            jax.ShapeDtypeStruct((_B, _N, _N), jnp.int32),
            jax.ShapeDtypeStruct((_B, _SLOTP), f32),
        ],
        in_specs=[vmem] * 13 + [smem, smem],
        out_specs=[vmem] * 4,
    )(x, Wphi1, bphi1.reshape(1, _H1), Wphi2, bphi2.reshape(1, _H2),
      Wpsi1, bpsi1.reshape(1, _H1), Wpsi2, bpsi2.reshape(1, _H2),
      Wobj, bobj.reshape(1, _DOUT), i0, i1, nobj, nrel)

    table = interaction_feature.reshape(_B * _NN, _DE)
    gathered = _sc_gather_fn()(slot.reshape(_B, _NN), table)

    rel = pl.pallas_call(
        _rel_body,
        out_shape=jax.ShapeDtypeStruct((_B * _SLOTP, _DOUT), f32),
    )(gathered, Wrel, brel.reshape(1, _DOUT), vmask.reshape(_B * _SLOTP, 1))

    rel_padded = rel.reshape(_B, _SLOTP, _DOUT)[:, :_MAXE]
    return score, nemb.reshape(_B, _N, _DOUT), rel_padded


# speedup vs baseline: 4.7715x; 1.0123x over previous
import functools

import jax
import jax.numpy as jnp
from jax import lax
from jax.experimental import pallas as pl
from jax.experimental.pallas import tpu as pltpu
from jax.experimental.pallas import tpu_sc as plsc

_B, _N, _DN, _DE, _H1, _H2, _DOUT, _MAXP, _MAXE = 8, 64, 2048, 1024, 512, 256, 512, 32, 172
_NN = _N * _N
_SLOTP = 192
_L = 16
_NW = 32
_WPS = _NW // _B
_RPW = _SLOTP // _WPS


def _nemb_body(x_ref, wobj_ref, bobj_ref, nemb_ref):
    nemb_ref[...] = jnp.dot(x_ref[...], wobj_ref[...]) + bobj_ref[...]


def _dense_body(x_ref, wphi1_ref, bphi1_ref, wphi2_ref, bphi2_ref,
                wpsi1_ref, bpsi1_ref, wpsi2_ref, bpsi2_ref,
                i0_ref, i1_ref, nobj_ref, nrel_ref,
                score_ref, slot_ref, vmask_ref):
    f32 = jnp.float32
    x = x_ref[...]
    h = jnp.maximum(jnp.dot(x, wphi1_ref[...]) + bphi1_ref[...], 0.0)
    phi = jnp.dot(h, wphi2_ref[...]) + bphi2_ref[...]
    h = jnp.maximum(jnp.dot(x, wpsi1_ref[...]) + bpsi1_ref[...], 0.0)
    psi = jnp.dot(h, wpsi2_ref[...]) + bpsi2_ref[...]

    row_i = lax.broadcasted_iota(jnp.int32, (_N, _N), 0)
    col_j = lax.broadcasted_iota(jnp.int32, (_N, _N), 1)
    upper = (row_i <= col_j).astype(f32)
    lows = (col_j < row_i).astype(f32)
    it_node = lax.broadcasted_iota(jnp.int32, (_N, _MAXP), 0)
    it_pair = lax.broadcasted_iota(jnp.int32, (_N, _MAXP), 1)
    iota_s = lax.broadcasted_iota(jnp.int32, (1, _SLOTP), 1).astype(f32)
    dn = (((1,), (1,)), ((), ()))

    for b in range(_B):
        phib = phi[b * _N:(b + 1) * _N]
        psib = psi[b * _N:(b + 1) * _N]
        score = jax.nn.sigmoid(lax.dot_general(phib, psib, dn))
        score_ref[b] = score
        nobj = nobj_ref[b]
        nrel = nrel_ref[b]
        adj = (score > 0.5) & (row_i < nobj) & (col_j < nobj)
        i0b = jnp.broadcast_to(i0_ref[b:b + 1, :], (_N, _MAXP))
        i1b = jnp.broadcast_to(i1_ref[b:b + 1, :], (_N, _MAXP))
        pvalid = it_pair < nrel
        m0 = (it_node == i0b)
        m1 = (it_node == i1b)
        fa = lax.dot_general((m0 & pvalid).astype(f32), m1.astype(f32), dn)
        fb = lax.dot_general((m1 & pvalid).astype(f32), m0.astype(f32), dn)
        adjf = jnp.where(adj | (fa + fb > 0.5), 1.0, 0.0).astype(f32)
        rowcum = jnp.dot(adjf, upper)
        rowtot = rowcum[:, _N - 1:_N]
        prefix = jnp.dot(lows, rowtot)
        rank = prefix + rowcum - 1.0
        validf = (adjf > 0.5) & (rank < float(_MAXE))
        slot_ref[b] = jnp.where(validf, rank, float(_MAXE)).astype(jnp.int32)
        total = jnp.sum(adjf)
        vmask_ref[b:b + 1, :] = (iota_s < total).astype(f32)


def _rel_body(g_ref, wrel_ref, brel_ref, vmask_ref, out_ref):
    out_ref[...] = (jnp.dot(g_ref[...], wrel_ref[...]) + brel_ref[...]) * vmask_ref[...]


@functools.cache
def _sc_gather_fn():
    mesh = plsc.VectorSubcoreMesh(core_axis_name="c", subcore_axis_name="s",
                                  num_cores=2, num_subcores=16)

    @functools.partial(
        pl.kernel,
        out_type=jax.ShapeDtypeStruct((_B * _SLOTP, _DE), jnp.float32),
        mesh=mesh,
        scratch_types=[
            pltpu.VMEM((_NN,), jnp.int32),
            pltpu.VMEM((_SLOTP,), jnp.int32),
            pltpu.VMEM((_RPW,), jnp.int32),
            pltpu.VMEM((_RPW, _DE), jnp.float32),
            pltpu.SemaphoreType.DMA,
        ],
        compiler_params=pltpu.CompilerParams(needs_layout_passes=False),
    )
    def sc_gather(slot_hbm, table_hbm, out_hbm, slot_v, eidx_v, idx_v, rows_v, sem):
        wid = lax.axis_index("s") * 2 + lax.axis_index("c")
        scene = wid // _WPS
        sub = wid % _WPS
        pltpu.sync_copy(slot_hbm.at[scene], slot_v)
        zero = jnp.zeros((_L,), jnp.int32)
        for k in range(_SLOTP // _L):
            eidx_v[pl.ds(k * _L, _L)] = zero
        lane = lax.iota(jnp.int32, _L)

        def scat(e, carry):
            base = e * _L
            chunk = slot_v[pl.ds(base, _L)]
            plsc.store_scatter(eidx_v, [chunk], lane + base)
            return carry

        lax.fori_loop(0, _NN // _L, scat, 0)
        off = jnp.full((_L,), scene * _NN, jnp.int32)
        base = sub * _RPW
        for k in range(_RPW // _L):
            idx_v[pl.ds(k * _L, _L)] = eidx_v[pl.ds(base + k * _L, _L)] + off
        pltpu.async_copy(table_hbm.at[idx_v], rows_v, sem).wait()
        pltpu.sync_copy(rows_v, out_hbm.at[pl.ds(scene * _SLOTP + base, _RPW)])

    return sc_gather


def kernel(concatenated_node_features, interaction_feature, num_obj, num_relation,
           object_pairs, Wphi1, bphi1, Wphi2, bphi2, Wpsi1, bpsi1, Wpsi2, bpsi2,
           Wobj, bobj, Wrel, brel):
    f32 = jnp.float32
    x = concatenated_node_features.reshape(_B * _N, _DN)
    i0 = object_pairs[..., 0].astype(jnp.int32)
    i1 = object_pairs[..., 1].astype(jnp.int32)
    nobj = num_obj.astype(jnp.int32)
    nrel = num_relation.astype(jnp.int32)

    smem = pl.BlockSpec(memory_space=pltpu.SMEM)
    vmem = pl.BlockSpec(memory_space=pltpu.VMEM)
    score, slot, vmask = pl.pallas_call(
        _dense_body,
        out_shape=[
            jax.ShapeDtypeStruct((_B, _N, _N), f32),
            jax.ShapeDtypeStruct((_B, _N, _N), jnp.int32),
            jax.ShapeDtypeStruct((_B, _SLOTP), f32),
        ],
        in_specs=[vmem] * 11 + [smem, smem],
        out_specs=[vmem] * 3,
    )(x, Wphi1, bphi1.reshape(1, _H1), Wphi2, bphi2.reshape(1, _H2),
      Wpsi1, bpsi1.reshape(1, _H1), Wpsi2, bpsi2.reshape(1, _H2),
      i0, i1, nobj, nrel)

    table = interaction_feature.reshape(_B * _NN, _DE)
    gathered = _sc_gather_fn()(slot.reshape(_B, _NN), table)

    nemb = pl.pallas_call(
        _nemb_body,
        out_shape=jax.ShapeDtypeStruct((_B * _N, _DOUT), f32),
    )(x, Wobj, bobj.reshape(1, _DOUT))

    rel = pl.pallas_call(
        _rel_body,
        out_shape=jax.ShapeDtypeStruct((_B * _SLOTP, _DOUT), f32),
    )(gathered, Wrel, brel.reshape(1, _DOUT), vmask.reshape(_B * _SLOTP, 1))

    rel_padded = rel.reshape(_B, _SLOTP, _DOUT)[:, :_MAXE]
    return score, nemb.reshape(_B, _N, _DOUT), rel_padded
